# Initial kernel scaffold; baseline (speedup 1.0000x reference)
import jax
import jax.numpy as jnp
from jax.experimental import pallas as pl


def kernel(x, edge_index, edge_type, edge_time, lambda_sym, beta, Wf, bf, Wr1, br1, Wr2, br2, W0, b0, W1, b1, W2, b2):
    raise NotImplementedError("write your pallas kernel here")


# speedup vs baseline: 6.0453x; 6.0453x over previous
import functools

import jax
import jax.numpy as jnp
from jax import lax
from jax.experimental import pallas as pl
from jax.experimental.pallas import tpu as pltpu
from jax.experimental.pallas import tpu_sc as plsc

_NC = 2
_NS = 16
_LANE = 128


def _emb_body(x_ref, wf_ref, bf_ref, o_ref):
    o_ref[...] = (
        jnp.dot(x_ref[...], wf_ref[...], preferred_element_type=jnp.float32)
        + bf_ref[...]
    )


def _edge_w_body(ett_ref, beta_ref, lam_ref, o_ref):
    tdim = ett_ref.shape[0]
    acc = ett_ref[0] * beta_ref[0, 0]
    for k in range(1, tdim):
        acc += ett_ref[k] * beta_ref[0, k]
    o_ref[...] = lam_ref[0, 0] * jnp.exp(-acc)


def _tables_body(h_ref, wt_ref, wb_ref, br_ref, ta_ref, tb_ref):
    h = h_ref[...]
    ta_ref[0] = jnp.dot(h, wt_ref[0], preferred_element_type=jnp.float32)
    tb_ref[0] = (
        jnp.dot(h, wb_ref[0], preferred_element_type=jnp.float32) + br_ref[0]
    )


def _tables_sum_body(p_ref, wt_ref, wb_ref, br_ref, ta_ref, tb_ref, h_ref):
    h = p_ref[0] + p_ref[1]
    h_ref[...] = h
    ta_ref[0] = jnp.dot(h, wt_ref[0], preferred_element_type=jnp.float32)
    tb_ref[0] = (
        jnp.dot(h, wb_ref[0], preferred_element_type=jnp.float32) + br_ref[0]
    )


def _final_body(h0_ref, h1_ref, p2_ref, w0_ref, b0_ref, w1_ref, b1_ref,
                w2_ref, b2_ref, o_ref):
    def lrelu(z):
        return jnp.where(z > 0, z, 0.01 * z)

    h2 = p2_ref[0] + p2_ref[1]
    acc = lrelu(jnp.dot(h0_ref[...], w0_ref[...],
                        preferred_element_type=jnp.float32) + b0_ref[...])
    acc += lrelu(jnp.dot(h1_ref[...], w1_ref[...],
                         preferred_element_type=jnp.float32) + b1_ref[...])
    acc += lrelu(jnp.dot(h2, w2_ref[...],
                         preferred_element_type=jnp.float32) + b2_ref[...])
    o_ref[...] = acc


def _sc_layer(ta, tb, gia, gib, dstr, wr, n_pad, H):
    rows_total = gia.shape[0]
    nw = _NC * _NS
    rw = rows_total // nw
    ch_rows = 8
    n_chunks = rw // ch_rows
    ch = ch_rows * _LANE
    nrows = n_pad // _NS

    mesh = plsc.VectorSubcoreMesh(core_axis_name="c", subcore_axis_name="s")

    @functools.partial(
        pl.kernel,
        out_type=jax.ShapeDtypeStruct((_NC, n_pad, H), jnp.float32),
        mesh=mesh,
        scratch_types=[
            pltpu.VMEM((rw, _LANE), jnp.int32),
            pltpu.VMEM((rw, _LANE), jnp.int32),
            pltpu.VMEM((rw, _LANE), jnp.int32),
            pltpu.VMEM((rw, _LANE), jnp.float32),
            pltpu.VMEM((ch, H), jnp.float32),
            pltpu.VMEM((ch, H), jnp.float32),
            pltpu.VMEM_SHARED((n_pad, H), jnp.float32),
            pltpu.SemaphoreType.DMA,
            pltpu.SemaphoreType.DMA,
        ],
        compiler_params=pltpu.CompilerParams(use_tc_tiling_on_sc=False),
    )
    def sck(gia_h, gib_h, dst_h, w_h, ta_h, tb_h, out_h,
            idxa, idxb, dstv, wv, bufa, bufb, accum, sema, semb):
        cid = lax.axis_index("c")
        sid = lax.axis_index("s")
        wid = cid * _NS + sid
        base = sid * nrows

        def zero_row(i, carry):
            z = jnp.zeros((16,), jnp.float32)
            bufa[i, pl.ds(0, 16)] = z
            bufa[i, pl.ds(16, 16)] = z
            return carry

        lax.fori_loop(0, nrows, zero_row, 0, unroll=4)
        pltpu.sync_copy(bufa.at[pl.ds(0, nrows)], accum.at[pl.ds(base, nrows)])
        plsc.subcore_barrier()

        erow0 = wid * rw
        pltpu.sync_copy(gia_h.at[pl.ds(erow0, rw)], idxa)
        pltpu.sync_copy(gib_h.at[pl.ds(erow0, rw)], idxb)
        pltpu.sync_copy(dst_h.at[pl.ds(erow0, rw)], dstv)
        pltpu.sync_copy(w_h.at[pl.ds(erow0, rw)], wv)

        def chunk_body(ci, carry):
            r0 = ci * ch_rows
            cps = []
            for j in range(ch_rows):
                cps.append(pltpu.async_copy(
                    ta_h.at[idxa.at[r0 + j]],
                    bufa.at[pl.ds(j * _LANE, _LANE)], sema))
                cps.append(pltpu.async_copy(
                    tb_h.at[idxb.at[r0 + j]],
                    bufb.at[pl.ds(j * _LANE, _LANE)], semb))
            for cp in cps:
                cp.wait()

            def group_body(g, carry2):
                gr = r0 + lax.shift_right_logical(g, 3)
                gl = lax.bitwise_and(g, 7) * 16
                w16 = wv[gr, pl.ds(gl, 16)]
                e0 = g * 16
                for k in range(16):
                    wvec = lax.gather(
                        w16, jnp.full((16, 1), k, jnp.int32),
                        lax.GatherDimensionNumbers(
                            offset_dims=(), collapsed_slice_dims=(0,),
                            start_index_map=(0,)),
                        slice_sizes=(1,),
                        mode=lax.GatherScatterMode.PROMISE_IN_BOUNDS)
                    e = e0 + k
                    lo = pl.ds(0, 16)
                    hi = pl.ds(16, 16)
                    bufa[e, lo] = (bufa[e, lo] + bufb[e, lo]) * wvec
                    bufa[e, hi] = (bufa[e, hi] + bufb[e, hi]) * wvec
                return carry2

            lax.fori_loop(0, ch // 16, group_body, 0)

            for j in range(ch_rows):
                pltpu.sync_copy(bufa.at[pl.ds(j * _LANE, _LANE)],
                                accum.at[dstv.at[r0 + j]], add=True)
            return carry

        lax.fori_loop(0, n_chunks, chunk_body, 0)

        plsc.subcore_barrier()
        pltpu.sync_copy(accum.at[pl.ds(base, nrows)], bufa.at[pl.ds(0, nrows)])
        pltpu.sync_copy(bufa.at[pl.ds(0, nrows)],
                        out_h.at[cid, pl.ds(base, nrows)])

    return sck(gia, gib, dstr, wr, ta, tb)


def kernel(x, edge_index, edge_type, edge_time, lambda_sym, beta, Wf, bf,
           Wr1, br1, Wr2, br2, W0, b0, W1, b1, W2, b2):
    N, in_dim = x.shape
    H = Wf.shape[1]
    R = Wr1.shape[0]
    E = edge_index.shape[1]
    out_dim = W0.shape[1]
    f32 = jnp.float32

    nb = 10
    bn = N // nb

    h0 = pl.pallas_call(
        _emb_body,
        grid=(nb,),
        in_specs=[
            pl.BlockSpec((bn, in_dim), lambda i: (i, 0)),
            pl.BlockSpec((in_dim, H), lambda i: (0, 0)),
            pl.BlockSpec((1, H), lambda i: (0, 0)),
        ],
        out_specs=pl.BlockSpec((bn, H), lambda i: (i, 0)),
        out_shape=jax.ShapeDtypeStruct((N, H), f32),
    )(x, Wf, bf.reshape(1, H))

    tdim = edge_time.shape[1]
    erows = E // _LANE
    ett = jnp.transpose(edge_time).reshape(tdim, erows, _LANE)
    w2d = pl.pallas_call(
        _edge_w_body,
        grid=(1,),
        in_specs=[
            pl.BlockSpec((tdim, erows, _LANE), lambda i: (0, 0, 0)),
            pl.BlockSpec((1, tdim), lambda i: (0, 0)),
            pl.BlockSpec((1, 1), lambda i: (0, 0)),
        ],
        out_specs=pl.BlockSpec((erows, _LANE), lambda i: (0, 0)),
        out_shape=jax.ShapeDtypeStruct((erows, _LANE), f32),
    )(ett, beta.reshape(1, tdim), lambda_sym)

    src = edge_index[0].astype(jnp.int32)
    dst = edge_index[1].astype(jnp.int32)
    ety = edge_type.astype(jnp.int32)
    gia = ety * N + src
    gib = ety * N + dst

    group = _NC * _NS * _LANE * 8
    e_pad = ((E + group - 1) // group) * group
    n_pad = ((N + 8 * _NS - 1) // (8 * _NS)) * (8 * _NS)
    padn = e_pad - E
    zi = jnp.zeros((padn,), jnp.int32)
    zf = jnp.zeros((padn,), f32)
    gia_p = jnp.concatenate([gia, zi]).reshape(-1, _LANE)
    gib_p = jnp.concatenate([gib, zi]).reshape(-1, _LANE)
    dst_p = jnp.concatenate([dst, zi]).reshape(-1, _LANE)
    w_p = jnp.concatenate([w2d, zf.reshape(-1, _LANE)])

    tbl_specs = dict(
        grid=(nb, R),
        out_shape=[
            jax.ShapeDtypeStruct((R, N, H), f32),
            jax.ShapeDtypeStruct((R, N, H), f32),
        ],
    )
    wt_spec = pl.BlockSpec((1, H, H), lambda i, r: (r, 0, 0))
    br_spec = pl.BlockSpec((1, 1, H), lambda i, r: (r, 0, 0))
    t_out = [
        pl.BlockSpec((1, bn, H), lambda i, r: (r, i, 0)),
        pl.BlockSpec((1, bn, H), lambda i, r: (r, i, 0)),
    ]

    wt1 = Wr1[:, :H, :]
    wb1 = Wr1[:, H:, :]
    ta1, tb1 = pl.pallas_call(
        _tables_body,
        in_specs=[
            pl.BlockSpec((bn, H), lambda i, r: (i, 0)),
            wt_spec, wt_spec, br_spec,
        ],
        out_specs=t_out,
        **tbl_specs,
    )(h0, wt1, wb1, br1.reshape(R, 1, H))

    p1 = _sc_layer(ta1.reshape(R * N, H), tb1.reshape(R * N, H),
                   gia_p, gib_p, dst_p, w_p, n_pad, H)

    wt2 = Wr2[:, :H, :]
    wb2 = Wr2[:, H:, :]
    ta2, tb2, h1 = pl.pallas_call(
        _tables_sum_body,
        grid=(nb, R),
        in_specs=[
            pl.BlockSpec((_NC, bn, H), lambda i, r: (0, i, 0)),
            wt_spec, wt_spec, br_spec,
        ],
        out_specs=t_out + [pl.BlockSpec((bn, H), lambda i, r: (i, 0))],
        out_shape=[
            jax.ShapeDtypeStruct((R, N, H), f32),
            jax.ShapeDtypeStruct((R, N, H), f32),
            jax.ShapeDtypeStruct((N, H), f32),
        ],
    )(p1, wt2, wb2, br2.reshape(R, 1, H))

    p2 = _sc_layer(ta2.reshape(R * N, H), tb2.reshape(R * N, H),
                   gia_p, gib_p, dst_p, w_p, n_pad, H)

    out = pl.pallas_call(
        _final_body,
        grid=(nb,),
        in_specs=[
            pl.BlockSpec((bn, H), lambda i: (i, 0)),
            pl.BlockSpec((bn, H), lambda i: (i, 0)),
            pl.BlockSpec((_NC, bn, H), lambda i: (0, i, 0)),
            pl.BlockSpec((H, out_dim), lambda i: (0, 0)),
            pl.BlockSpec((1, out_dim), lambda i: (0, 0)),
            pl.BlockSpec((H, out_dim), lambda i: (0, 0)),
            pl.BlockSpec((1, out_dim), lambda i: (0, 0)),
            pl.BlockSpec((H, out_dim), lambda i: (0, 0)),
            pl.BlockSpec((1, out_dim), lambda i: (0, 0)),
        ],
        out_specs=pl.BlockSpec((bn, out_dim), lambda i: (i, 0)),
        out_shape=jax.ShapeDtypeStruct((N, out_dim), f32),
    )(h0, h1, p2, W0, b0.reshape(1, out_dim), W1, b1.reshape(1, out_dim),
      W2, b2.reshape(1, out_dim))

    return out


# speedup vs baseline: 8.3450x; 1.3804x over previous
import functools

import jax
import jax.numpy as jnp
from jax import lax
from jax.experimental import pallas as pl
from jax.experimental.pallas import tpu as pltpu
from jax.experimental.pallas import tpu_sc as plsc

_NC = 2
_NS = 16
_LANE = 128


def _emb_body(x_ref, wf_ref, bf_ref, o_ref):
    o_ref[...] = (
        jnp.dot(x_ref[...], wf_ref[...], preferred_element_type=jnp.float32)
        + bf_ref[...]
    )


def _edge_w_body(ett_ref, beta_ref, lam_ref, o_ref):
    tdim = ett_ref.shape[0]
    acc = ett_ref[0] * beta_ref[0, 0]
    for k in range(1, tdim):
        acc += ett_ref[k] * beta_ref[0, k]
    o_ref[...] = lam_ref[0, 0] * jnp.exp(-acc)


def _tables_body(h_ref, wt_ref, wb_ref, br_ref, ta_ref, tb_ref):
    h = h_ref[...]
    ta_ref[0] = jnp.dot(h, wt_ref[0], preferred_element_type=jnp.float32)
    tb_ref[0] = (
        jnp.dot(h, wb_ref[0], preferred_element_type=jnp.float32) + br_ref[0]
    )


def _tables_sum_body(p_ref, wt_ref, wb_ref, br_ref, ta_ref, tb_ref, h_ref):
    h = p_ref[0] + p_ref[1]
    h_ref[...] = h
    ta_ref[0] = jnp.dot(h, wt_ref[0], preferred_element_type=jnp.float32)
    tb_ref[0] = (
        jnp.dot(h, wb_ref[0], preferred_element_type=jnp.float32) + br_ref[0]
    )


def _final_body(h0_ref, h1_ref, p2_ref, w0_ref, b0_ref, w1_ref, b1_ref,
                w2_ref, b2_ref, o_ref):
    def lrelu(z):
        return jnp.where(z > 0, z, 0.01 * z)

    h2 = p2_ref[0] + p2_ref[1]
    acc = lrelu(jnp.dot(h0_ref[...], w0_ref[...],
                        preferred_element_type=jnp.float32) + b0_ref[...])
    acc += lrelu(jnp.dot(h1_ref[...], w1_ref[...],
                         preferred_element_type=jnp.float32) + b1_ref[...])
    acc += lrelu(jnp.dot(h2, w2_ref[...],
                         preferred_element_type=jnp.float32) + b2_ref[...])
    o_ref[...] = acc


def _sc_layer(ta, tb, gia, gib, dstr, wr, n_pad, H):
    rows_total = gia.shape[0]
    nw = _NC * _NS
    rw = rows_total // nw
    ch_rows = 4
    n_chunks = rw // ch_rows
    ch = ch_rows * _LANE
    nrows = n_pad // _NS

    mesh = plsc.VectorSubcoreMesh(core_axis_name="c", subcore_axis_name="s")

    @functools.partial(
        pl.kernel,
        out_type=jax.ShapeDtypeStruct((_NC, n_pad, H), jnp.float32),
        mesh=mesh,
        scratch_types=[
            pltpu.VMEM((rw, _LANE), jnp.int32),
            pltpu.VMEM((rw, _LANE), jnp.int32),
            pltpu.VMEM((rw, _LANE), jnp.int32),
            pltpu.VMEM((rw, _LANE), jnp.float32),
            pltpu.VMEM((ch, H), jnp.float32),
            pltpu.VMEM((ch, H), jnp.float32),
            pltpu.VMEM((ch, H), jnp.float32),
            pltpu.VMEM((ch, H), jnp.float32),
            pltpu.VMEM_SHARED((n_pad, H), jnp.float32),
            pltpu.SemaphoreType.DMA,
            pltpu.SemaphoreType.DMA,
            pltpu.SemaphoreType.DMA,
            pltpu.SemaphoreType.DMA,
        ],
        compiler_params=pltpu.CompilerParams(use_tc_tiling_on_sc=False),
    )
    def sck(gia_h, gib_h, dst_h, w_h, ta_h, tb_h, out_h,
            idxa, idxb, dstv, wv, bufa0, bufa1, bufb0, bufb1, accum,
            sg0, sg1, ss0, ss1):
        cid = lax.axis_index("c")
        sid = lax.axis_index("s")
        wid = cid * _NS + sid
        base = sid * nrows
        bufa = (bufa0, bufa1)
        bufb = (bufb0, bufb1)
        sg = (sg0, sg1)
        ss = (ss0, ss1)

        def zero_row(i, carry):
            z = jnp.zeros((16,), jnp.float32)
            bufa0[i, pl.ds(0, 16)] = z
            bufa0[i, pl.ds(16, 16)] = z
            return carry

        lax.fori_loop(0, ch, zero_row, 0, unroll=4)
        pltpu.sync_copy(bufa0.at[pl.ds(0, ch)], accum.at[pl.ds(base, ch)])
        rem = nrows - ch
        pltpu.sync_copy(bufa0.at[pl.ds(0, rem)],
                        accum.at[pl.ds(base + ch, rem)])
        plsc.subcore_barrier()

        erow0 = wid * rw
        pltpu.sync_copy(gia_h.at[pl.ds(erow0, rw)], idxa)
        pltpu.sync_copy(gib_h.at[pl.ds(erow0, rw)], idxb)
        pltpu.sync_copy(dst_h.at[pl.ds(erow0, rw)], dstv)
        pltpu.sync_copy(w_h.at[pl.ds(erow0, rw)], wv)

        def fire_gathers(ci, s):
            r0 = ci * ch_rows
            cps = []
            for j in range(ch_rows):
                cps.append(pltpu.async_copy(
                    ta_h.at[idxa.at[r0 + j]],
                    bufa[s].at[pl.ds(j * _LANE, _LANE)], sg[s]))
                cps.append(pltpu.async_copy(
                    tb_h.at[idxb.at[r0 + j]],
                    bufb[s].at[pl.ds(j * _LANE, _LANE)], sg[s]))
            return cps

        def compute_chunk(ci, s):
            ba = bufa[s]
            bb = bufb[s]
            r0 = ci * ch_rows

            def group_body(g, carry2):
                gr = r0 + lax.shift_right_logical(g, 3)
                gl = lax.bitwise_and(g, 7) * 16
                w16 = wv[gr, pl.ds(gl, 16)]
                e0 = g * 16
                for k in range(16):
                    wvec = lax.gather(
                        w16, jnp.full((16, 1), k, jnp.int32),
                        lax.GatherDimensionNumbers(
                            offset_dims=(), collapsed_slice_dims=(0,),
                            start_index_map=(0,)),
                        slice_sizes=(1,),
                        mode=lax.GatherScatterMode.PROMISE_IN_BOUNDS)
                    e = e0 + k
                    lo = pl.ds(0, 16)
                    hi = pl.ds(16, 16)
                    ba[e, lo] = (ba[e, lo] + bb[e, lo]) * wvec
                    ba[e, hi] = (ba[e, hi] + bb[e, hi]) * wvec
                return carry2

            lax.fori_loop(0, ch // 16, group_body, 0)

        def fire_scatters(ci, s):
            r0 = ci * ch_rows
            cps = []
            for j in range(ch_rows):
                cps.append(pltpu.async_copy(
                    bufa[s].at[pl.ds(j * _LANE, _LANE)],
                    accum.at[dstv.at[r0 + j]], ss[s], add=True))
            return cps

        pend_scatter = [None, None]
        pend_gather = [None, None]
        pend_gather[0] = fire_gathers(0, 0)
        for ci in range(n_chunks):
            s = ci % 2
            o = 1 - s
            if ci + 1 < n_chunks:
                if pend_scatter[o] is not None:
                    for cp in pend_scatter[o]:
                        cp.wait()
                    pend_scatter[o] = None
                pend_gather[o] = fire_gathers(ci + 1, o)
            for cp in pend_gather[s]:
                cp.wait()
            compute_chunk(ci, s)
            pend_scatter[s] = fire_scatters(ci, s)
        for s in range(2):
            if pend_scatter[s] is not None:
                for cp in pend_scatter[s]:
                    cp.wait()

        plsc.subcore_barrier()
        pltpu.sync_copy(accum.at[pl.ds(base, ch)], bufa0.at[pl.ds(0, ch)])
        pltpu.sync_copy(bufa0.at[pl.ds(0, ch)],
                        out_h.at[cid, pl.ds(base, ch)])
        pltpu.sync_copy(accum.at[pl.ds(base + ch, rem)],
                        bufa1.at[pl.ds(0, rem)])
        pltpu.sync_copy(bufa1.at[pl.ds(0, rem)],
                        out_h.at[cid, pl.ds(base + ch, rem)])

    return sck(gia, gib, dstr, wr, ta, tb)


def kernel(x, edge_index, edge_type, edge_time, lambda_sym, beta, Wf, bf,
           Wr1, br1, Wr2, br2, W0, b0, W1, b1, W2, b2):
    N, in_dim = x.shape
    H = Wf.shape[1]
    R = Wr1.shape[0]
    E = edge_index.shape[1]
    out_dim = W0.shape[1]
    f32 = jnp.float32

    nb = 10
    bn = N // nb

    h0 = pl.pallas_call(
        _emb_body,
        grid=(nb,),
        in_specs=[
            pl.BlockSpec((bn, in_dim), lambda i: (i, 0)),
            pl.BlockSpec((in_dim, H), lambda i: (0, 0)),
            pl.BlockSpec((1, H), lambda i: (0, 0)),
        ],
        out_specs=pl.BlockSpec((bn, H), lambda i: (i, 0)),
        out_shape=jax.ShapeDtypeStruct((N, H), f32),
    )(x, Wf, bf.reshape(1, H))

    tdim = edge_time.shape[1]
    erows = E // _LANE
    ett = jnp.transpose(edge_time).reshape(tdim, erows, _LANE)
    w2d = pl.pallas_call(
        _edge_w_body,
        grid=(1,),
        in_specs=[
            pl.BlockSpec((tdim, erows, _LANE), lambda i: (0, 0, 0)),
            pl.BlockSpec((1, tdim), lambda i: (0, 0)),
            pl.BlockSpec((1, 1), lambda i: (0, 0)),
        ],
        out_specs=pl.BlockSpec((erows, _LANE), lambda i: (0, 0)),
        out_shape=jax.ShapeDtypeStruct((erows, _LANE), f32),
    )(ett, beta.reshape(1, tdim), lambda_sym)

    src = edge_index[0].astype(jnp.int32)
    dst = edge_index[1].astype(jnp.int32)
    ety = edge_type.astype(jnp.int32)
    gia = ety * N + src
    gib = ety * N + dst

    group = _NC * _NS * _LANE * 8
    e_pad = ((E + group - 1) // group) * group
    n_pad = ((N + 8 * _NS - 1) // (8 * _NS)) * (8 * _NS)
    padn = e_pad - E
    zi = jnp.zeros((padn,), jnp.int32)
    zf = jnp.zeros((padn,), f32)
    gia_p = jnp.concatenate([gia, zi]).reshape(-1, _LANE)
    gib_p = jnp.concatenate([gib, zi]).reshape(-1, _LANE)
    dst_p = jnp.concatenate([dst, zi]).reshape(-1, _LANE)
    w_p = jnp.concatenate([w2d, zf.reshape(-1, _LANE)])

    tbl_specs = dict(
        grid=(nb, R),
        out_shape=[
            jax.ShapeDtypeStruct((R, N, H), f32),
            jax.ShapeDtypeStruct((R, N, H), f32),
        ],
    )
    wt_spec = pl.BlockSpec((1, H, H), lambda i, r: (r, 0, 0))
    br_spec = pl.BlockSpec((1, 1, H), lambda i, r: (r, 0, 0))
    t_out = [
        pl.BlockSpec((1, bn, H), lambda i, r: (r, i, 0)),
        pl.BlockSpec((1, bn, H), lambda i, r: (r, i, 0)),
    ]

    wt1 = Wr1[:, :H, :]
    wb1 = Wr1[:, H:, :]
    ta1, tb1 = pl.pallas_call(
        _tables_body,
        in_specs=[
            pl.BlockSpec((bn, H), lambda i, r: (i, 0)),
            wt_spec, wt_spec, br_spec,
        ],
        out_specs=t_out,
        **tbl_specs,
    )(h0, wt1, wb1, br1.reshape(R, 1, H))

    p1 = _sc_layer(ta1.reshape(R * N, H), tb1.reshape(R * N, H),
                   gia_p, gib_p, dst_p, w_p, n_pad, H)

    wt2 = Wr2[:, :H, :]
    wb2 = Wr2[:, H:, :]
    ta2, tb2, h1 = pl.pallas_call(
        _tables_sum_body,
        grid=(nb, R),
        in_specs=[
            pl.BlockSpec((_NC, bn, H), lambda i, r: (0, i, 0)),
            wt_spec, wt_spec, br_spec,
        ],
        out_specs=t_out + [pl.BlockSpec((bn, H), lambda i, r: (i, 0))],
        out_shape=[
            jax.ShapeDtypeStruct((R, N, H), f32),
            jax.ShapeDtypeStruct((R, N, H), f32),
            jax.ShapeDtypeStruct((N, H), f32),
        ],
    )(p1, wt2, wb2, br2.reshape(R, 1, H))

    p2 = _sc_layer(ta2.reshape(R * N, H), tb2.reshape(R * N, H),
                   gia_p, gib_p, dst_p, w_p, n_pad, H)

    out = pl.pallas_call(
        _final_body,
        grid=(nb,),
        in_specs=[
            pl.BlockSpec((bn, H), lambda i: (i, 0)),
            pl.BlockSpec((bn, H), lambda i: (i, 0)),
            pl.BlockSpec((_NC, bn, H), lambda i: (0, i, 0)),
            pl.BlockSpec((H, out_dim), lambda i: (0, 0)),
            pl.BlockSpec((1, out_dim), lambda i: (0, 0)),
            pl.BlockSpec((H, out_dim), lambda i: (0, 0)),
            pl.BlockSpec((1, out_dim), lambda i: (0, 0)),
            pl.BlockSpec((H, out_dim), lambda i: (0, 0)),
            pl.BlockSpec((1, out_dim), lambda i: (0, 0)),
        ],
        out_specs=pl.BlockSpec((bn, out_dim), lambda i: (i, 0)),
        out_shape=jax.ShapeDtypeStruct((N, out_dim), f32),
    )(h0, h1, p2, W0, b0.reshape(1, out_dim), W1, b1.reshape(1, out_dim),
      W2, b2.reshape(1, out_dim))

    return out
